# Initial kernel scaffold; baseline (speedup 1.0000x reference)
import jax
import jax.numpy as jnp
from jax.experimental import pallas as pl


def kernel(x, edge_index, edge_weight, node2graph, W, b, prelu_a):
    raise NotImplementedError("write your pallas kernel here")


# speedup vs baseline: 6.6374x; 6.6374x over previous
import functools

import jax
import jax.numpy as jnp
from jax import lax
from jax.experimental import pallas as pl
from jax.experimental.pallas import tpu as pltpu
from jax.experimental.pallas import tpu_sc as plsc

_N = 10000
_E = 320000
_DIN = 128
_DOUT = 64
_B = 64

_NC = 2
_NS = 16
_NW = _NC * _NS
_EPW = _E // _NW
_CK = 400
_NCH = _EPW // _CK
_RPT = 632

_mesh = plsc.VectorSubcoreMesh(core_axis_name="c", subcore_axis_name="s")


@functools.partial(
    pl.kernel,
    out_type=[
        jax.ShapeDtypeStruct((_NC, _N), jnp.float32),
        jax.ShapeDtypeStruct((_NC, _N), jnp.float32),
    ],
    mesh=_mesh,
    scratch_types=[
        pltpu.VMEM((_EPW,), jnp.int32),
        pltpu.VMEM((_EPW,), jnp.int32),
        pltpu.VMEM((_EPW,), jnp.float32),
        pltpu.VMEM_SHARED((_N,), jnp.float32),
        pltpu.VMEM_SHARED((_N,), jnp.float32),
    ],
)
def _deg_kernel(src_hbm, dst_hbm, dego_hbm, degi_hbm,
                srcv, dstv, onesv, dego_s, degi_s):
    cid = lax.axis_index("c")
    sid = lax.axis_index("s")
    base = (cid * _NS + sid) * _EPW

    def zloop(i, _):
        onesv[pl.ds(i * 16, 16)] = jnp.zeros((16,), jnp.float32)
        return 0
    lax.fori_loop(0, _EPW // 16, zloop, 0)

    @pl.when(sid == 0)
    def _():
        pltpu.sync_copy(onesv, dego_s)
        pltpu.sync_copy(onesv, degi_s)

    def oloop(i, _):
        onesv[pl.ds(i * 16, 16)] = jnp.ones((16,), jnp.float32)
        return 0
    lax.fori_loop(0, _EPW // 16, oloop, 0)

    plsc.subcore_barrier()
    pltpu.sync_copy(src_hbm.at[pl.ds(base, _EPW)], srcv)
    pltpu.sync_copy(dst_hbm.at[pl.ds(base, _EPW)], dstv)
    pltpu.sync_copy(onesv, dego_s.at[srcv], add=True)
    pltpu.sync_copy(onesv, degi_s.at[dstv], add=True)
    plsc.subcore_barrier()

    @pl.when(sid == 0)
    def _():
        pltpu.sync_copy(dego_s, dego_hbm.at[cid])
        pltpu.sync_copy(degi_s, degi_hbm.at[cid])


@functools.partial(
    pl.kernel,
    out_type=jax.ShapeDtypeStruct((_NC * _N, _DOUT), jnp.float32),
    mesh=_mesh,
    scratch_types=[
        pltpu.VMEM((_CK,), jnp.int32),
        pltpu.VMEM((_CK,), jnp.int32),
        pltpu.VMEM((_CK,), jnp.float32),
        pltpu.VMEM((_CK, _DOUT), jnp.float32),
        pltpu.VMEM_SHARED((_N, _DOUT), jnp.float32),
        pltpu.SemaphoreType.DMA,
    ],
    compiler_params=pltpu.CompilerParams(use_tc_tiling_on_sc=False,
                                         needs_layout_passes=False),
)
def _edge_kernel(featp_hbm, src_hbm, dst_hbm, ew_hbm, agg_hbm,
                 srcv, dstv, ewv, rowsv, agg_s, sem):
    cid = lax.axis_index("c")
    sid = lax.axis_index("s")
    base = (cid * _NS + sid) * _EPW

    def zloop(i, _):
        for c in range(_DOUT // 16):
            rowsv[i, pl.ds(c * 16, 16)] = jnp.zeros((16,), jnp.float32)
        return 0
    lax.fori_loop(0, _CK, zloop, 0)
    row0 = pl.multiple_of(jnp.minimum(sid * _RPT, _N - _RPT), 8)
    pltpu.sync_copy(rowsv, agg_s.at[pl.ds(row0, _CK)])
    pltpu.sync_copy(rowsv.at[pl.ds(0, _RPT - _CK)],
                    agg_s.at[pl.ds(row0 + _CK, _RPT - _CK)])
    plsc.subcore_barrier()

    def chunk(j, _):
        off = base + j * _CK
        pltpu.sync_copy(src_hbm.at[pl.ds(off, _CK)], srcv)
        pltpu.sync_copy(dst_hbm.at[pl.ds(off, _CK)], dstv)
        pltpu.sync_copy(ew_hbm.at[pl.ds(off, _CK)], ewv)
        pltpu.async_copy(featp_hbm.at[srcv], rowsv, sem).wait()

        def srow(i, _):
            wv = plsc.load_gather(ewv, [jnp.full((16,), i, jnp.int32)])
            for c in range(_DOUT // 16):
                sl = pl.ds(c * 16, 16)
                rowsv[i, sl] = rowsv[i, sl] * wv
            return 0
        lax.fori_loop(0, _CK, srow, 0)
        pltpu.sync_copy(rowsv, agg_s.at[dstv], add=True)
        return 0
    lax.fori_loop(0, _NCH, chunk, 0)

    plsc.subcore_barrier()
    pltpu.sync_copy(agg_s.at[pl.ds(row0, _CK)],
                    agg_hbm.at[pl.ds(cid * _N + row0, _CK)])
    pltpu.sync_copy(agg_s.at[pl.ds(row0 + _CK, _RPT - _CK)],
                    agg_hbm.at[pl.ds(cid * _N + row0 + _CK, _RPT - _CK)])


def _feat_body(x_ref, w_ref, dego_ref, out_ref):
    deg = dego_ref[0] + dego_ref[1]
    norm = lax.rsqrt(jnp.maximum(deg, 1.0))
    xw = jnp.dot(x_ref[...], w_ref[...], preferred_element_type=jnp.float32)
    out_ref[...] = xw * norm


_feat_call = pl.pallas_call(
    _feat_body,
    out_shape=jax.ShapeDtypeStruct((_N, _DOUT), jnp.float32),
)


def _epi_body(x_ref, w_ref, agg_ref, degi_ref, n2g_ref, b_ref, a_ref,
              h_out, pool_out, anc_out):
    a = a_ref[0, 0]
    bias = b_ref[...]
    agg = agg_ref[pl.ds(0, _N), :] + agg_ref[pl.ds(_N, _N), :]
    degi = degi_ref[0] + degi_ref[1]
    ndst = lax.rsqrt(jnp.maximum(degi, 1.0))
    h = agg * ndst + bias
    hp = jnp.maximum(h, 0.0) + a * jnp.minimum(h, 0.0)
    hn = jnp.sqrt(jnp.sum(hp * hp, axis=1, keepdims=True))
    h_out[...] = hp / jnp.maximum(hn, 1e-12)

    n2g = n2g_ref[...]
    gids = lax.broadcasted_iota(jnp.int32, (_N, _B), 1)
    oh = (n2g == gids).astype(jnp.float32)
    ones_col = jnp.ones((_N, 1), jnp.float32)
    cdims = (((0,), (0,)), ((), ()))
    pool_sum = lax.dot_general(oh, hp, cdims, preferred_element_type=jnp.float32)
    cnt = lax.dot_general(oh, ones_col, cdims, preferred_element_type=jnp.float32)
    pool = pool_sum / jnp.maximum(cnt, 1.0)
    pn = jnp.sqrt(jnp.sum(pool * pool, axis=1, keepdims=True))
    pool_out[...] = pool / jnp.maximum(pn, 1e-12)

    less = (n2g < gids).astype(jnp.float32)
    cntl = lax.dot_general(ones_col, less, cdims, preferred_element_type=jnp.float32)
    aidx = jnp.minimum(cntl, float(_N - 1)).astype(jnp.int32)
    nio = lax.broadcasted_iota(jnp.int32, (_N, _B), 0)
    aoh = (nio == aidx).astype(jnp.float32)
    ax = lax.dot_general(aoh, x_ref[...], cdims, preferred_element_type=jnp.float32)
    ao = jnp.dot(ax, w_ref[...], preferred_element_type=jnp.float32) + bias
    aop = jnp.maximum(ao, 0.0) + a * jnp.minimum(ao, 0.0)
    an = jnp.sqrt(jnp.sum(aop * aop, axis=1, keepdims=True))
    anc_out[...] = aop / jnp.maximum(an, 1e-12)


_epi_call = pl.pallas_call(
    _epi_body,
    out_shape=[
        jax.ShapeDtypeStruct((_N, _DOUT), jnp.float32),
        jax.ShapeDtypeStruct((_B, _DOUT), jnp.float32),
        jax.ShapeDtypeStruct((_B, _DOUT), jnp.float32),
    ],
)


def kernel(x, edge_index, edge_weight, node2graph, W, b, prelu_a):
    src = edge_index[0]
    dst = edge_index[1]
    dego, degi = _deg_kernel(src, dst)
    featp = _feat_call(x, W, dego.reshape(_NC, _N, 1))
    agg2 = _edge_kernel(featp, src, dst, edge_weight)
    h, pool, anc = _epi_call(x, W, agg2, degi.reshape(_NC, _N, 1),
                             node2graph.reshape(_N, 1),
                             b.reshape(1, _DOUT),
                             jnp.asarray(prelu_a, jnp.float32).reshape(1, 1))
    return h, pool, anc


# speedup vs baseline: 9.0353x; 1.3613x over previous
import functools

import jax
import jax.numpy as jnp
from jax import lax
from jax.experimental import pallas as pl
from jax.experimental.pallas import tpu as pltpu
from jax.experimental.pallas import tpu_sc as plsc

_N = 10000
_E = 320000
_DIN = 128
_DOUT = 64
_B = 64

_NC = 2
_NS = 16
_NW = _NC * _NS
_EPW = _E // _NW
_CK = 250
_NCH = _EPW // _CK
_SEGS = ((0, 248), (248, 248), (496, 136))
_RPT = 632

_mesh = plsc.VectorSubcoreMesh(core_axis_name="c", subcore_axis_name="s")


@functools.partial(
    pl.kernel,
    out_type=[
        jax.ShapeDtypeStruct((_NC, _N), jnp.float32),
        jax.ShapeDtypeStruct((_NC, _N), jnp.float32),
    ],
    mesh=_mesh,
    scratch_types=[
        pltpu.VMEM((_EPW,), jnp.int32),
        pltpu.VMEM((_EPW,), jnp.int32),
        pltpu.VMEM((_EPW,), jnp.float32),
        pltpu.VMEM_SHARED((_N,), jnp.float32),
        pltpu.VMEM_SHARED((_N,), jnp.float32),
    ],
)
def _deg_kernel(src_hbm, dst_hbm, dego_hbm, degi_hbm,
                srcv, dstv, onesv, dego_s, degi_s):
    cid = lax.axis_index("c")
    sid = lax.axis_index("s")
    base = (cid * _NS + sid) * _EPW

    def zloop(i, _):
        onesv[pl.ds(i * 16, 16)] = jnp.zeros((16,), jnp.float32)
        return 0
    lax.fori_loop(0, _EPW // 16, zloop, 0)

    @pl.when(sid == 0)
    def _():
        pltpu.sync_copy(onesv, dego_s)
        pltpu.sync_copy(onesv, degi_s)

    def oloop(i, _):
        onesv[pl.ds(i * 16, 16)] = jnp.ones((16,), jnp.float32)
        return 0
    lax.fori_loop(0, _EPW // 16, oloop, 0)

    plsc.subcore_barrier()
    pltpu.sync_copy(src_hbm.at[pl.ds(base, _EPW)], srcv)
    pltpu.sync_copy(dst_hbm.at[pl.ds(base, _EPW)], dstv)
    pltpu.sync_copy(onesv, dego_s.at[srcv], add=True)
    pltpu.sync_copy(onesv, degi_s.at[dstv], add=True)
    plsc.subcore_barrier()

    @pl.when(sid == 0)
    def _():
        pltpu.sync_copy(dego_s, dego_hbm.at[cid])
        pltpu.sync_copy(degi_s, degi_hbm.at[cid])


@functools.partial(
    pl.kernel,
    out_type=jax.ShapeDtypeStruct((_NC * _N, _DOUT), jnp.float32),
    mesh=_mesh,
    scratch_types=[
        pltpu.VMEM((_NCH, _CK), jnp.int32),
        pltpu.VMEM((_NCH, _CK), jnp.int32),
        pltpu.VMEM((_NCH, _CK), jnp.float32),
        pltpu.VMEM((_CK, _DOUT), jnp.float32),
        pltpu.VMEM((_CK, _DOUT), jnp.float32),
        pltpu.VMEM_SHARED((_N, _DOUT), jnp.float32),
        pltpu.SemaphoreType.DMA,
        pltpu.SemaphoreType.DMA,
        pltpu.SemaphoreType.DMA,
        pltpu.SemaphoreType.DMA,
    ],
    compiler_params=pltpu.CompilerParams(use_tc_tiling_on_sc=False,
                                         needs_layout_passes=False),
)
def _edge_kernel(featp_hbm, src_hbm, dst_hbm, ew_hbm, agg_hbm,
                 srcb, dstb, ewb, rows0, rows1, agg_s,
                 semg0, semg1, sems0, sems1):
    cid = lax.axis_index("c")
    sid = lax.axis_index("s")
    r0 = (cid * _NS + sid) * _NCH
    pltpu.sync_copy(src_hbm.at[pl.ds(r0, _NCH)], srcb)
    pltpu.sync_copy(dst_hbm.at[pl.ds(r0, _NCH)], dstb)
    pltpu.sync_copy(ew_hbm.at[pl.ds(r0, _NCH)], ewb)

    def zloop(i, _):
        for c in range(_DOUT // 16):
            rows0[i, pl.ds(c * 16, 16)] = jnp.zeros((16,), jnp.float32)
        return 0
    lax.fori_loop(0, _CK, zloop, 0)
    row0 = pl.multiple_of(jnp.minimum(sid * _RPT, _N - _RPT), 8)
    for o, ln in _SEGS:
        pltpu.sync_copy(rows0.at[pl.ds(0, ln)], agg_s.at[pl.ds(row0 + o, ln)])
    pltpu.async_copy(featp_hbm.at[srcb.at[0]], rows0, semg0)
    plsc.subcore_barrier()

    def scale(rows, j):
        def srow(i, _):
            wv = plsc.load_gather(ewb, [jnp.full((16,), j, jnp.int32),
                                        jnp.full((16,), i, jnp.int32)])
            for c in range(_DOUT // 16):
                sl = pl.ds(c * 16, 16)
                rows[i, sl] = rows[i, sl] * wv
            return 0
        lax.fori_loop(0, _CK, srow, 0)

    def body(g, _):
        c0 = 2 * g
        c1 = c0 + 1
        @pl.when(g > 0)
        def _():
            pltpu.make_async_copy(rows1, agg_s.at[dstb.at[c1]], sems1).wait()
        pltpu.async_copy(featp_hbm.at[srcb.at[c1]], rows1, semg1)
        pltpu.make_async_copy(featp_hbm.at[srcb.at[c0]], rows0, semg0).wait()
        scale(rows0, c0)
        pltpu.async_copy(rows0, agg_s.at[dstb.at[c0]], sems0, add=True)
        pltpu.make_async_copy(featp_hbm.at[srcb.at[c1]], rows1, semg1).wait()
        scale(rows1, c1)
        pltpu.async_copy(rows1, agg_s.at[dstb.at[c1]], sems1, add=True)
        pltpu.make_async_copy(rows0, agg_s.at[dstb.at[c0]], sems0).wait()
        @pl.when(g < _NCH // 2 - 1)
        def _():
            pltpu.async_copy(featp_hbm.at[srcb.at[c0 + 2]], rows0, semg0)
        return 0
    lax.fori_loop(0, _NCH // 2, body, 0)
    pltpu.make_async_copy(rows1, agg_s.at[dstb.at[_NCH - 1]], sems1).wait()

    plsc.subcore_barrier()
    for o, ln in _SEGS:
        pltpu.sync_copy(agg_s.at[pl.ds(row0 + o, ln)],
                        agg_hbm.at[pl.ds(cid * _N + row0 + o, ln)])


def _feat_body(x_ref, w_ref, dego_ref, out_ref):
    deg = dego_ref[0] + dego_ref[1]
    norm = lax.rsqrt(jnp.maximum(deg, 1.0))
    xw = jnp.dot(x_ref[...], w_ref[...], preferred_element_type=jnp.float32)
    out_ref[...] = xw * norm


_feat_call = pl.pallas_call(
    _feat_body,
    out_shape=jax.ShapeDtypeStruct((_N, _DOUT), jnp.float32),
)


def _epi_body(x_ref, w_ref, agg_ref, degi_ref, n2g_ref, b_ref, a_ref,
              h_out, pool_out, anc_out):
    a = a_ref[0, 0]
    bias = b_ref[...]
    agg = agg_ref[pl.ds(0, _N), :] + agg_ref[pl.ds(_N, _N), :]
    degi = degi_ref[0] + degi_ref[1]
    ndst = lax.rsqrt(jnp.maximum(degi, 1.0))
    h = agg * ndst + bias
    hp = jnp.maximum(h, 0.0) + a * jnp.minimum(h, 0.0)
    hn = jnp.sqrt(jnp.sum(hp * hp, axis=1, keepdims=True))
    h_out[...] = hp / jnp.maximum(hn, 1e-12)

    n2g = n2g_ref[...]
    gids = lax.broadcasted_iota(jnp.int32, (_N, _B), 1)
    oh = (n2g == gids).astype(jnp.float32)
    ones_col = jnp.ones((_N, 1), jnp.float32)
    cdims = (((0,), (0,)), ((), ()))
    pool_sum = lax.dot_general(oh, hp, cdims, preferred_element_type=jnp.float32)
    cnt = lax.dot_general(oh, ones_col, cdims, preferred_element_type=jnp.float32)
    pool = pool_sum / jnp.maximum(cnt, 1.0)
    pn = jnp.sqrt(jnp.sum(pool * pool, axis=1, keepdims=True))
    pool_out[...] = pool / jnp.maximum(pn, 1e-12)

    less = (n2g < gids).astype(jnp.float32)
    cntl = lax.dot_general(ones_col, less, cdims, preferred_element_type=jnp.float32)
    aidx = jnp.minimum(cntl, float(_N - 1)).astype(jnp.int32)
    nio = lax.broadcasted_iota(jnp.int32, (_N, _B), 0)
    aoh = (nio == aidx).astype(jnp.float32)
    ax = lax.dot_general(aoh, x_ref[...], cdims, preferred_element_type=jnp.float32)
    ao = jnp.dot(ax, w_ref[...], preferred_element_type=jnp.float32) + bias
    aop = jnp.maximum(ao, 0.0) + a * jnp.minimum(ao, 0.0)
    an = jnp.sqrt(jnp.sum(aop * aop, axis=1, keepdims=True))
    anc_out[...] = aop / jnp.maximum(an, 1e-12)


_epi_call = pl.pallas_call(
    _epi_body,
    out_shape=[
        jax.ShapeDtypeStruct((_N, _DOUT), jnp.float32),
        jax.ShapeDtypeStruct((_B, _DOUT), jnp.float32),
        jax.ShapeDtypeStruct((_B, _DOUT), jnp.float32),
    ],
)


def kernel(x, edge_index, edge_weight, node2graph, W, b, prelu_a):
    src = edge_index[0]
    dst = edge_index[1]
    dego, degi = _deg_kernel(src, dst)
    featp = _feat_call(x, W, dego.reshape(_NC, _N, 1))
    agg2 = _edge_kernel(featp, src.reshape(_E // _CK, _CK),
                        dst.reshape(_E // _CK, _CK),
                        edge_weight.reshape(_E // _CK, _CK))
    h, pool, anc = _epi_call(x, W, agg2, degi.reshape(_NC, _N, 1),
                             node2graph.reshape(_N, 1),
                             b.reshape(1, _DOUT),
                             jnp.asarray(prelu_a, jnp.float32).reshape(1, 1))
    return h, pool, anc


# speedup vs baseline: 9.1618x; 1.0140x over previous
import functools

import jax
import jax.numpy as jnp
from jax import lax
from jax.experimental import pallas as pl
from jax.experimental.pallas import tpu as pltpu
from jax.experimental.pallas import tpu_sc as plsc

_N = 10000
_E = 320000
_DIN = 128
_DOUT = 64
_B = 64

_NC = 2
_NS = 16
_NW = _NC * _NS
_EPW = _E // _NW
_CK = 250
_NCH = _EPW // _CK
_SEGS = ((0, 248), (248, 248), (496, 136))
_RPT = 632

_mesh = plsc.VectorSubcoreMesh(core_axis_name="c", subcore_axis_name="s")


@functools.partial(
    pl.kernel,
    out_type=[
        jax.ShapeDtypeStruct((_NC, _N), jnp.float32),
        jax.ShapeDtypeStruct((_NC, _N), jnp.float32),
    ],
    mesh=_mesh,
    scratch_types=[
        pltpu.VMEM((_EPW,), jnp.int32),
        pltpu.VMEM((_EPW,), jnp.int32),
        pltpu.VMEM((_EPW,), jnp.float32),
        pltpu.VMEM_SHARED((_N,), jnp.float32),
        pltpu.VMEM_SHARED((_N,), jnp.float32),
    ],
)
def _deg_kernel(src_hbm, dst_hbm, dego_hbm, degi_hbm,
                srcv, dstv, onesv, dego_s, degi_s):
    cid = lax.axis_index("c")
    sid = lax.axis_index("s")
    base = (cid * _NS + sid) * _EPW

    def zloop(i, _):
        onesv[pl.ds(i * 16, 16)] = jnp.zeros((16,), jnp.float32)
        return 0
    lax.fori_loop(0, _EPW // 16, zloop, 0)

    @pl.when(sid == 0)
    def _():
        pltpu.sync_copy(onesv, dego_s)
        pltpu.sync_copy(onesv, degi_s)

    def oloop(i, _):
        onesv[pl.ds(i * 16, 16)] = jnp.ones((16,), jnp.float32)
        return 0
    lax.fori_loop(0, _EPW // 16, oloop, 0)

    plsc.subcore_barrier()
    pltpu.sync_copy(src_hbm.at[pl.ds(base, _EPW)], srcv)
    pltpu.sync_copy(dst_hbm.at[pl.ds(base, _EPW)], dstv)
    pltpu.sync_copy(onesv, dego_s.at[srcv], add=True)
    pltpu.sync_copy(onesv, degi_s.at[dstv], add=True)
    plsc.subcore_barrier()

    @pl.when(sid == 0)
    def _():
        pltpu.sync_copy(dego_s, dego_hbm.at[cid])
        pltpu.sync_copy(degi_s, degi_hbm.at[cid])


@functools.partial(
    pl.kernel,
    out_type=jax.ShapeDtypeStruct((_NC * _N, _DOUT), jnp.float32),
    mesh=_mesh,
    scratch_types=[
        pltpu.VMEM((_NCH, _CK), jnp.int32),
        pltpu.VMEM((_NCH, _CK), jnp.int32),
        pltpu.VMEM((_NCH, _CK), jnp.float32),
        pltpu.VMEM((_CK, _DOUT), jnp.float32),
        pltpu.VMEM((_CK, _DOUT), jnp.float32),
        pltpu.VMEM_SHARED((_N, _DOUT), jnp.float32),
        pltpu.SemaphoreType.DMA,
        pltpu.SemaphoreType.DMA,
        pltpu.SemaphoreType.DMA,
        pltpu.SemaphoreType.DMA,
    ],
    compiler_params=pltpu.CompilerParams(use_tc_tiling_on_sc=False,
                                         needs_layout_passes=False),
)
def _edge_kernel(featp_hbm, src_hbm, dst_hbm, ew_hbm, agg_hbm,
                 srcb, dstb, ewb, rows0, rows1, agg_s,
                 semg0, semg1, sems0, sems1):
    cid = lax.axis_index("c")
    sid = lax.axis_index("s")
    r0 = (cid * _NS + sid) * _NCH
    pltpu.sync_copy(src_hbm.at[pl.ds(r0, _NCH)], srcb)
    pltpu.sync_copy(dst_hbm.at[pl.ds(r0, _NCH)], dstb)
    pltpu.sync_copy(ew_hbm.at[pl.ds(r0, _NCH)], ewb)

    def zloop(i, _):
        for c in range(_DOUT // 16):
            rows0[i, pl.ds(c * 16, 16)] = jnp.zeros((16,), jnp.float32)
        return 0
    lax.fori_loop(0, _CK, zloop, 0)
    row0 = pl.multiple_of(jnp.minimum(sid * _RPT, _N - _RPT), 8)
    for o, ln in _SEGS:
        pltpu.sync_copy(rows0.at[pl.ds(0, ln)], agg_s.at[pl.ds(row0 + o, ln)])
    pltpu.async_copy(featp_hbm.at[srcb.at[0]], rows0, semg0)
    plsc.subcore_barrier()

    def scale(rows, j):
        jv = jnp.full((16,), j, jnp.int32)

        def srow(q, _):
            for u in range(2):
                i = q * 2 + u
                wv = plsc.load_gather(ewb, [jv, jnp.full((16,), i, jnp.int32)])
                for c in range(_DOUT // 16):
                    sl = pl.ds(c * 16, 16)
                    rows[i, sl] = rows[i, sl] * wv
            return 0
        lax.fori_loop(0, _CK // 2, srow, 0)

    def body(g, _):
        c0 = 2 * g
        c1 = c0 + 1
        @pl.when(g > 0)
        def _():
            pltpu.make_async_copy(rows1, agg_s.at[dstb.at[c1]], sems1).wait()
        pltpu.async_copy(featp_hbm.at[srcb.at[c1]], rows1, semg1)
        pltpu.make_async_copy(featp_hbm.at[srcb.at[c0]], rows0, semg0).wait()
        scale(rows0, c0)
        pltpu.async_copy(rows0, agg_s.at[dstb.at[c0]], sems0, add=True)
        pltpu.make_async_copy(featp_hbm.at[srcb.at[c1]], rows1, semg1).wait()
        scale(rows1, c1)
        pltpu.async_copy(rows1, agg_s.at[dstb.at[c1]], sems1, add=True)
        pltpu.make_async_copy(rows0, agg_s.at[dstb.at[c0]], sems0).wait()
        @pl.when(g < _NCH // 2 - 1)
        def _():
            pltpu.async_copy(featp_hbm.at[srcb.at[c0 + 2]], rows0, semg0)
        return 0
    lax.fori_loop(0, _NCH // 2, body, 0)
    pltpu.make_async_copy(rows1, agg_s.at[dstb.at[_NCH - 1]], sems1).wait()

    plsc.subcore_barrier()
    for o, ln in _SEGS:
        pltpu.sync_copy(agg_s.at[pl.ds(row0 + o, ln)],
                        agg_hbm.at[pl.ds(cid * _N + row0 + o, ln)])


def _feat_body(x_ref, w_ref, dego_ref, out_ref):
    deg = dego_ref[0] + dego_ref[1]
    norm = lax.rsqrt(jnp.maximum(deg, 1.0))
    xw = jnp.dot(x_ref[...], w_ref[...], preferred_element_type=jnp.float32)
    out_ref[...] = xw * norm


_feat_call = pl.pallas_call(
    _feat_body,
    out_shape=jax.ShapeDtypeStruct((_N, _DOUT), jnp.float32),
)


def _epi_body(x_ref, w_ref, agg_ref, degi_ref, n2g_ref, b_ref, a_ref,
              h_out, pool_out, anc_out):
    a = a_ref[0, 0]
    bias = b_ref[...]
    agg = agg_ref[pl.ds(0, _N), :] + agg_ref[pl.ds(_N, _N), :]
    degi = degi_ref[0] + degi_ref[1]
    ndst = lax.rsqrt(jnp.maximum(degi, 1.0))
    h = agg * ndst + bias
    hp = jnp.maximum(h, 0.0) + a * jnp.minimum(h, 0.0)
    hn = jnp.sqrt(jnp.sum(hp * hp, axis=1, keepdims=True))
    h_out[...] = hp / jnp.maximum(hn, 1e-12)

    n2g = n2g_ref[...]
    gids = lax.broadcasted_iota(jnp.int32, (_N, _B), 1)
    oh = (n2g == gids).astype(jnp.float32)
    ones_col = jnp.ones((_N, 1), jnp.float32)
    cdims = (((0,), (0,)), ((), ()))
    pool_sum = lax.dot_general(oh, hp, cdims, preferred_element_type=jnp.float32)
    cnt = lax.dot_general(oh, ones_col, cdims, preferred_element_type=jnp.float32)
    pool = pool_sum / jnp.maximum(cnt, 1.0)
    pn = jnp.sqrt(jnp.sum(pool * pool, axis=1, keepdims=True))
    pool_out[...] = pool / jnp.maximum(pn, 1e-12)

    less = (n2g < gids).astype(jnp.float32)
    cntl = lax.dot_general(ones_col, less, cdims, preferred_element_type=jnp.float32)
    aidx = jnp.minimum(cntl, float(_N - 1)).astype(jnp.int32)
    nio = lax.broadcasted_iota(jnp.int32, (_N, _B), 0)
    aoh = (nio == aidx).astype(jnp.float32)
    ax = lax.dot_general(aoh, x_ref[...], cdims, preferred_element_type=jnp.float32)
    ao = jnp.dot(ax, w_ref[...], preferred_element_type=jnp.float32) + bias
    aop = jnp.maximum(ao, 0.0) + a * jnp.minimum(ao, 0.0)
    an = jnp.sqrt(jnp.sum(aop * aop, axis=1, keepdims=True))
    anc_out[...] = aop / jnp.maximum(an, 1e-12)


_epi_call = pl.pallas_call(
    _epi_body,
    out_shape=[
        jax.ShapeDtypeStruct((_N, _DOUT), jnp.float32),
        jax.ShapeDtypeStruct((_B, _DOUT), jnp.float32),
        jax.ShapeDtypeStruct((_B, _DOUT), jnp.float32),
    ],
)


def kernel(x, edge_index, edge_weight, node2graph, W, b, prelu_a):
    src = edge_index[0]
    dst = edge_index[1]
    dego, degi = _deg_kernel(src, dst)
    featp = _feat_call(x, W, dego.reshape(_NC, _N, 1))
    agg2 = _edge_kernel(featp, src.reshape(_E // _CK, _CK),
                        dst.reshape(_E // _CK, _CK),
                        edge_weight.reshape(_E // _CK, _CK))
    h, pool, anc = _epi_call(x, W, agg2, degi.reshape(_NC, _N, 1),
                             node2graph.reshape(_N, 1),
                             b.reshape(1, _DOUT),
                             jnp.asarray(prelu_a, jnp.float32).reshape(1, 1))
    return h, pool, anc
